# baseline (speedup 1.0000x reference)
import jax
import jax.numpy as jnp
from jax.experimental import pallas as pl

N = 10000
E = 320000
D = 128
H = 32
NG = 128
C = 2
EMB = 96
F32 = jnp.float32


def _tc_pred(g_s, g_a, qW1, qb1, qg1, qbt1, qW2, qb2, qg2, qbt2, qW3, qb3):
    def bn(x, g, b):
        mu = jnp.mean(x, axis=0, keepdims=True)
        var = jnp.mean((x - mu) * (x - mu), axis=0, keepdims=True)
        return g * (x - mu) / jnp.sqrt(var + 1e-5) + b

    def body(gs_r, ga_r, w1_r, b1_r, g1_r, t1_r, w2_r, b2_r, g2_r, t2_r,
             w3_r, b3_r, zs_o, za_o):
        for src_g, z_o in ((gs_r[...], zs_o), (ga_r[...], za_o)):
            h = jnp.maximum(
                bn(jnp.dot(src_g, w1_r[...]) + b1_r[...],
                   g1_r[...], t1_r[...]), 0.0)
            h = jnp.maximum(
                bn(jnp.dot(h, w2_r[...]) + b2_r[...],
                   g2_r[...], t2_r[...]), 0.0)
            z_o[...] = jnp.dot(h, w3_r[...]) + b3_r[...]

    return pl.pallas_call(
        body,
        in_specs=[pl.BlockSpec((NG, EMB), lambda: (0, 0)),
                  pl.BlockSpec((NG, EMB), lambda: (0, 0)),
                  pl.BlockSpec((EMB, EMB), lambda: (0, 0)),
                  pl.BlockSpec((1, EMB), lambda: (0, 0)),
                  pl.BlockSpec((1, EMB), lambda: (0, 0)),
                  pl.BlockSpec((1, EMB), lambda: (0, 0)),
                  pl.BlockSpec((EMB, H), lambda: (0, 0)),
                  pl.BlockSpec((1, H), lambda: (0, 0)),
                  pl.BlockSpec((1, H), lambda: (0, 0)),
                  pl.BlockSpec((1, H), lambda: (0, 0)),
                  pl.BlockSpec((H, C), lambda: (0, 0)),
                  pl.BlockSpec((1, C), lambda: (0, 0))],
        out_specs=[pl.BlockSpec((NG, C), lambda: (0, 0)),
                   pl.BlockSpec((NG, C), lambda: (0, 0))],
        out_shape=[jax.ShapeDtypeStruct((NG, C), F32),
                   jax.ShapeDtypeStruct((NG, C), F32)],
    )(g_s, g_a, qW1, qb1.reshape(1, -1), qg1.reshape(1, -1),
      qbt1.reshape(1, -1), qW2, qb2.reshape(1, -1), qg2.reshape(1, -1),
      qbt2.reshape(1, -1), qW3, qb3.reshape(1, -1))


def kernel(x, edge_index, batch, W1, b1, W2, b2, W3, b3,
           pW1, pb1, pg1, pbt1, pW2, pb2,
           aW1, ab1, ag1, abt1, aW2, ab2,
           qW1, qb1, qg1, qbt1, qW2, qb2, qg2, qbt2, qW3, qb3):
    src, dst = edge_index[0], edge_index[1]

    def layer(h, W, b, ew):
        m = h[src]
        if ew is not None:
            m = m * ew
        agg = jax.ops.segment_sum(m, dst, num_segments=N)
        return jax.nn.relu((h @ W) + (agg @ W) + b)

    def encode(xin, ew):
        h1 = layer(xin, W1, b1, ew)
        h2 = layer(h1, W2, b2, ew)
        h3 = layer(h2, W3, b3, ew)
        node_emb = jnp.concatenate([h1, h2, h3], axis=1)
        g = jax.ops.segment_sum(node_emb, batch, num_segments=NG)
        return g, node_emb

    def _bn(xx, g, b):
        mu = jnp.mean(xx, axis=0, keepdims=True)
        var = jnp.var(xx, axis=0, keepdims=True)
        return g * (xx - mu) / jnp.sqrt(var + 1e-5) + b

    def prompt(node_emb, Wa, ba, g1, bt1, Wb, bb):
        e = jnp.concatenate([node_emb[src], node_emb[dst]], axis=1)
        h = jax.nn.relu(_bn(e @ Wa + ba, g1, bt1))
        w = jax.nn.sigmoid(h @ Wb + bb)
        return jnp.nan_to_num(w, nan=0.5)

    _, node_emb = encode(x, None)
    sw = prompt(node_emb, pW1, pb1, pg1, pbt1, pW2, pb2)
    aw = prompt(node_emb, aW1, ab1, ag1, abt1, aW2, ab2)
    h_s, _ = encode(x, sw)
    h_a, _ = encode(x, aw)

    z_s, z_a = _tc_pred(h_s, h_a, qW1, qb1, qg1, qbt1, qW2, qb2, qg2,
                        qbt2, qW3, qb3)
    return (h_s, z_s, h_a, z_a)


# speedup vs baseline: 2.4485x; 2.4485x over previous
import functools

import jax
import jax.numpy as jnp
from jax import lax
from jax.experimental import pallas as pl
from jax.experimental.pallas import tpu as pltpu
from jax.experimental.pallas import tpu_sc as plsc

N = 10000
E = 320000
D = 128
H = 32
NG = 128
C = 2
EMB = 96
F32 = jnp.float32

NCORE = 2
NSUB = 16
NW = NCORE * NSUB
FT = 128
WROW = 64


def _sc_mesh():
    return plsc.VectorSubcoreMesh(
        core_axis_name="c", subcore_axis_name="s",
        num_cores=NCORE, num_subcores=NSUB)


def _make_sorted_segsum(FU, n_out, n_edges, CH, ew_mode):
    ec = n_edges // NW
    nch = ec // CH
    assert ec % CH == 0 and CH % 16 == 0
    nv = FU // 16
    hv = nv // 2
    RW = 2 * WROW

    scratch = [
        pltpu.VMEM((CH,), jnp.int32),
        pltpu.VMEM((CH,), jnp.int32),
        pltpu.VMEM((CH, FT), F32),
        pltpu.VMEM(((RW + 1) * FU,), F32),
        pltpu.VMEM((8 * FU,), F32),
        pltpu.VMEM((16,), jnp.int32),
        pltpu.SemaphoreType.DMA,
    ]
    if ew_mode:
        scratch += [pltpu.VMEM((CH,), F32)]
    if ew_mode == 2:
        scratch += [pltpu.VMEM((CH,), F32)]

    def body(*refs):
        if ew_mode == 2:
            (table, gidx, sid, ews, ewa, out, parts,
             gi_v, si_v, rows_v, win, pf, bv, sem, ews_v, ewa_v) = refs
        elif ew_mode == 1:
            (table, gidx, sid, ews, out, parts,
             gi_v, si_v, rows_v, win, pf, bv, sem, ews_v) = refs
        else:
            (table, gidx, sid, out, parts,
             gi_v, si_v, rows_v, win, pf, bv, sem) = refs
        c = lax.axis_index("c")
        s = lax.axis_index("s")
        wid = c * NSUB + s

        def zrow(r, _):
            win[pl.ds(r * 16, 16)] = jnp.zeros((16,), F32)
            return 0
        lax.fori_loop(0, (RW + 1) * FU // 16, zrow, 0)


        def zpf(r, _):
            pf[pl.ds(r * 16, 16)] = jnp.zeros((16,), F32)
            return 0
        lax.fori_loop(0, 8 * FU // 16, zpf, 0)

        pltpu.sync_copy(sid.at[pl.ds(wid * ec, 16)], bv)
        lo = bv[pl.ds(0, 16)][0]
        nxt_off = jnp.minimum((wid + 1) * ec, n_edges - 16)
        pltpu.sync_copy(sid.at[pl.ds(nxt_off, 16)], bv)
        lo_next = jnp.where(wid == NW - 1, n_out, bv[pl.ds(0, 16)][0])
        swb0 = jnp.where(wid == 0, 0, lo + 1)

        def flush(swb):
            half = jnp.remainder(swb - swb0, RW) * FU
            cp = pltpu.make_async_copy(
                win.at[pl.ds(half, WROW * FU)],
                out.at[pl.ds(swb * FU, WROW * FU)], sem)
            cp.start()
            cp.wait()
            def rz(r, _):
                win[pl.ds(half + r * 16, 16)] = jnp.zeros((16,), F32)
                return 0
            lax.fori_loop(0, WROW * FU // 16, rz, 0)
            return swb + WROW

        zacc = tuple(jnp.zeros((16,), F32) for _ in range(nv))

        def chunk(i, carry):
            cur, swb, acc = carry
            base = wid * ec + i * CH
            pltpu.sync_copy(gidx.at[pl.ds(base, CH)], gi_v)
            pltpu.sync_copy(sid.at[pl.ds(base, CH)], si_v)
            if ew_mode:
                pltpu.sync_copy(ews.at[pl.ds(base, CH)], ews_v)
            if ew_mode == 2:
                pltpu.sync_copy(ewa.at[pl.ds(base, CH)], ewa_v)
            pltpu.sync_copy(table.at[gi_v], rows_v)

            def group(g, carry):
                cur, swb, acc = carry
                sidv = si_v[pl.ds(g * 16, 16)]
                if ew_mode:
                    ewsv = ews_v[pl.ds(g * 16, 16)]
                if ew_mode == 2:
                    ewav = ewa_v[pl.ds(g * 16, 16)]
                e0 = g * 16
                for l in range(16):
                    d = sidv[l]
                    row = [rows_v[e0 + l, pl.ds(j * 16, 16)]
                           for j in range(nv)]
                    if ew_mode == 1:
                        w1 = ewsv[l]
                        row = [r * w1 for r in row]
                    elif ew_mode == 2:
                        w1 = ewsv[l]
                        w2 = ewav[l]
                        row = ([r * w1 for r in row[:hv]]
                               + [r * w2 for r in row[hv:]])
                    is_new = d != cur
                    acc = tuple(jnp.where(is_new, r, a + r)
                                for a, r in zip(acc, row))
                    cur = jnp.where(is_new, d, cur)
                    nfl = jnp.maximum(0, (cur - swb - RW) // WROW + 1)
                    swb = lax.fori_loop(0, nfl, lambda k, sb: flush(sb),
                                        swb)
                    rb = jnp.where(cur == lo, RW * FU,
                                   jnp.remainder(cur - swb0, RW) * FU)
                    for j in range(nv):
                        win[pl.ds(rb + j * 16, 16)] = acc[j]
                return cur, swb, acc

            return lax.fori_loop(0, CH // 16, group, (cur, swb, acc))

        cur, swb, acc = lax.fori_loop(0, nch, chunk, (lo, swb0, zacc))

        for j in range(nv):
            pf[pl.ds(j * 16, 16)] = win[pl.ds(RW * FU + j * 16, 16)]
        rbh = jnp.remainder(cur - swb0, RW) * FU
        for j in range(nv):
            pf[pl.ds(FU + j * 16, 16)] = win[pl.ds(rbh + j * 16, 16)]

        ndr = jnp.maximum(0, (lo_next - swb) // WROW)
        swb = lax.fori_loop(0, ndr, lambda k, sb: flush(sb), swb)
        pltpu.sync_copy(pf, parts.at[pl.ds(wid * 8 * FU, 8 * FU)])
        half = jnp.remainder(swb - swb0, RW) * FU
        cp = pltpu.make_async_copy(
            win.at[pl.ds(half, WROW * FU)],
            out.at[pl.ds((n_out + 8 + wid * WROW) * FU, WROW * FU)], sem)
        cp.start()
        cp.wait()

    return functools.partial(
        pl.kernel, body,
        out_type=(jax.ShapeDtypeStruct(((n_out + 8 + NW * WROW) * FU,),
                                       F32),
                  jax.ShapeDtypeStruct((NW * 8 * FU,), F32)),
        mesh=_sc_mesh(), scratch_types=scratch)


def _sorted_segsum(kern, n_out, n_edges, table, gidx, sid, ews=None,
                   ewa=None):
    ec = n_edges // NW
    args = [table, gidx, sid]
    if ews is not None:
        args.append(ews)
    if ewa is not None:
        args.append(ewa)
    outp, parts = kern()(*args)
    FU = parts.shape[0] // (NW * 8)
    outp = outp.reshape(-1, FU)
    parts = parts.reshape(NW, 8, FU)

    w = jnp.arange(NW, dtype=jnp.int32)
    lo = sid[w * ec]
    hi = sid[w * ec + ec - 1]
    lo_next = jnp.concatenate(
        [lo[1:], jnp.array([n_out], jnp.int32)])
    start = jnp.where(w == 0, 0, lo + 1)
    nfull = (lo_next - start) // WROW
    tail_base = start + nfull * WROW

    tails = outp[n_out + 8:].reshape(NW, WROW, FU)
    tid = tail_base[:, None] + jnp.arange(WROW, dtype=jnp.int32)[None, :]
    tvalid = (tid >= start[:, None]) & (tid < lo_next[:, None])
    tsafe = jnp.where(tvalid, tid, n_out).reshape(-1)
    out = outp[:n_out + 8]
    out = out.at[tsafe].set(tails.reshape(-1, FU), mode="drop",
                            unique_indices=False)

    ids = jnp.stack([lo, hi], axis=1).reshape(-1)
    badd = jnp.zeros((n_out + 8, FU), F32).at[ids].add(
        parts[:, :2].reshape(-1, FU), mode="drop")
    isb = jnp.zeros((n_out + 8,), jnp.bool_).at[ids].set(True, mode="drop")
    out = jnp.where(isb[:, None], badd, out)
    return out[:n_out]


def _tc_pred(g_s, g_a, qW1, qb1, qg1, qbt1, qW2, qb2, qg2, qbt2, qW3, qb3):
    def bn(x, g, b):
        mu = jnp.mean(x, axis=0, keepdims=True)
        var = jnp.mean((x - mu) * (x - mu), axis=0, keepdims=True)
        return g * (x - mu) / jnp.sqrt(var + 1e-5) + b

    def body(gs_r, ga_r, w1_r, b1_r, g1_r, t1_r, w2_r, b2_r, g2_r, t2_r,
             w3_r, b3_r, zs_o, za_o):
        for src_g, z_o in ((gs_r[...], zs_o), (ga_r[...], za_o)):
            h = jnp.maximum(
                bn(jnp.dot(src_g, w1_r[...]) + b1_r[...],
                   g1_r[...], t1_r[...]), 0.0)
            h = jnp.maximum(
                bn(jnp.dot(h, w2_r[...]) + b2_r[...],
                   g2_r[...], t2_r[...]), 0.0)
            z_o[...] = jnp.dot(h, w3_r[...]) + b3_r[...]

    return pl.pallas_call(
        body,
        in_specs=[pl.BlockSpec((NG, EMB), lambda: (0, 0)),
                  pl.BlockSpec((NG, EMB), lambda: (0, 0)),
                  pl.BlockSpec((EMB, EMB), lambda: (0, 0)),
                  pl.BlockSpec((1, EMB), lambda: (0, 0)),
                  pl.BlockSpec((1, EMB), lambda: (0, 0)),
                  pl.BlockSpec((1, EMB), lambda: (0, 0)),
                  pl.BlockSpec((EMB, H), lambda: (0, 0)),
                  pl.BlockSpec((1, H), lambda: (0, 0)),
                  pl.BlockSpec((1, H), lambda: (0, 0)),
                  pl.BlockSpec((1, H), lambda: (0, 0)),
                  pl.BlockSpec((H, C), lambda: (0, 0)),
                  pl.BlockSpec((1, C), lambda: (0, 0))],
        out_specs=[pl.BlockSpec((NG, C), lambda: (0, 0)),
                   pl.BlockSpec((NG, C), lambda: (0, 0))],
        out_shape=[jax.ShapeDtypeStruct((NG, C), F32),
                   jax.ShapeDtypeStruct((NG, C), F32)],
    )(g_s, g_a, qW1, qb1.reshape(1, -1), qg1.reshape(1, -1),
      qbt1.reshape(1, -1), qW2, qb2.reshape(1, -1), qg2.reshape(1, -1),
      qbt2.reshape(1, -1), qW3, qb3.reshape(1, -1))


NPOOL = 10240

_seg128 = _make_sorted_segsum(128, N, E, 400, 0)
_seg128w = _make_sorted_segsum(128, N, E, 400, 1)
_seg32 = _make_sorted_segsum(32, N, E, 400, 0)
_seg64d = _make_sorted_segsum(64, N, E, 400, 2)
_segpool = _make_sorted_segsum(96, NG + 1, NPOOL, 320, 0)


def _pad128(a):
    return jnp.concatenate(
        [a, jnp.zeros((a.shape[0], FT - a.shape[1]), F32)], axis=1)


def kernel(x, edge_index, batch, W1, b1, W2, b2, W3, b3,
           pW1, pb1, pg1, pbt1, pW2, pb2,
           aW1, ab1, ag1, abt1, aW2, ab2,
           qW1, qb1, qg1, qbt1, qW2, qb2, qg2, qbt2, qW3, qb3):
    src0, dst0 = edge_index[0], edge_index[1]
    dst, src, perm = lax.sort(
        [dst0, src0, jnp.arange(E, dtype=jnp.int32)], num_keys=1)

    pool_g = jnp.concatenate([jnp.arange(N, dtype=jnp.int32),
                              jnp.zeros((NPOOL - N,), jnp.int32)])
    pool_s = jnp.concatenate([batch.astype(jnp.int32),
                              jnp.full((NPOOL - N,), NG, jnp.int32)])

    def pool(ne):
        return _sorted_segsum(_segpool, NG + 1, NPOOL, _pad128(ne),
                              pool_g, pool_s)[:NG]

    def encode(ew_s, ew_a):
        if ew_s is None:
            a1 = _sorted_segsum(_seg128, N, E, x, src, dst)
            h1 = jax.nn.relu((x @ W1) + (a1 @ W1) + b1)
            a2 = _sorted_segsum(_seg32, N, E, _pad128(h1), src, dst)
            h2 = jax.nn.relu((h1 @ W2) + (a2 @ W2) + b2)
            a3 = _sorted_segsum(_seg32, N, E, _pad128(h2), src, dst)
            h3 = jax.nn.relu((h2 @ W3) + (a3 @ W3) + b3)
            return jnp.concatenate([h1, h2, h3], axis=1)
        a1s = _sorted_segsum(_seg128w, N, E, x, src, dst, ew_s)
        a1a = _sorted_segsum(_seg128w, N, E, x, src, dst, ew_a)
        h1s = jax.nn.relu((x @ W1) + (a1s @ W1) + b1)
        h1a = jax.nn.relu((x @ W1) + (a1a @ W1) + b1)
        t2 = _pad128(jnp.concatenate([h1s, h1a], axis=1))
        a2 = _sorted_segsum(_seg64d, N, E, t2, src, dst, ew_s, ew_a)
        h2s = jax.nn.relu((h1s @ W2) + (a2[:, :H] @ W2) + b2)
        h2a = jax.nn.relu((h1a @ W2) + (a2[:, H:] @ W2) + b2)
        t3 = _pad128(jnp.concatenate([h2s, h2a], axis=1))
        a3 = _sorted_segsum(_seg64d, N, E, t3, src, dst, ew_s, ew_a)
        h3s = jax.nn.relu((h2s @ W3) + (a3[:, :H] @ W3) + b3)
        h3a = jax.nn.relu((h2a @ W3) + (a3[:, H:] @ W3) + b3)
        ne_s = jnp.concatenate([h1s, h2s, h3s], axis=1)
        ne_a = jnp.concatenate([h1a, h2a, h3a], axis=1)
        return ne_s, ne_a

    node_emb = encode(None, None)

    def _bn(xx, g, b):
        mu = jnp.mean(xx, axis=0, keepdims=True)
        var = jnp.var(xx, axis=0, keepdims=True)
        return g * (xx - mu) / jnp.sqrt(var + 1e-5) + b

    e = jnp.concatenate([node_emb[src0], node_emb[dst0]], axis=1)

    def prompt(Wa, ba, g1, bt1, Wb, bb):
        hh = jax.nn.relu(_bn(e @ Wa + ba, g1, bt1))
        w = jax.nn.sigmoid(hh @ Wb + bb)
        return jnp.nan_to_num(w, nan=0.5)[:, 0]

    sw = prompt(pW1, pb1, pg1, pbt1, pW2, pb2)[perm]
    aw = prompt(aW1, ab1, ag1, abt1, aW2, ab2)[perm]

    ne_s, ne_a = encode(sw, aw)
    h_s = pool(ne_s)
    h_a = pool(ne_a)

    z_s, z_a = _tc_pred(h_s, h_a, qW1, qb1, qg1, qbt1, qW2, qb2, qg2,
                        qbt2, qW3, qb3)
    return (h_s, z_s, h_a, z_a)
